# baseline (speedup 1.0000x reference)
import functools

import jax
import jax.numpy as jnp
from jax import lax
from jax.experimental import pallas as pl
from jax.experimental.pallas import tpu as pltpu
from jax.experimental.pallas import tpu_sc as plsc

TOPK = 8


def _tc_body(x_ref, w_ref, b_ref, out_ref):
    logits = jnp.dot(x_ref[...], w_ref[...], preferred_element_type=jnp.float32)
    out_ref[...] = jax.nn.sigmoid(logits) + b_ref[...]


@functools.partial(jax.jit, static_argnames=("block_t",))
def _tc_biased_scores(x, w, bias_2d, block_t=2048):
    n, h = x.shape
    e = w.shape[1]
    return pl.pallas_call(
        _tc_body,
        grid=(n // block_t,),
        in_specs=[
            pl.BlockSpec((block_t, h), lambda i: (i, 0)),
            pl.BlockSpec((h, e), lambda i: (0, 0)),
            pl.BlockSpec((1, e), lambda i: (0, 0)),
        ],
        out_specs=pl.BlockSpec((block_t, e), lambda i: (i, 0)),
        out_shape=jax.ShapeDtypeStruct((n, e), jnp.float32),
    )(x, w, bias_2d)


def _make_sc_topk(n_tokens, n_experts):
    info = plsc.get_sparse_core_info()
    nc, ns, lanes = info.num_cores, info.num_subcores, info.num_lanes
    nw = nc * ns
    tok_per_w = n_tokens // nw
    nblk = n_experts // lanes
    mesh = plsc.VectorSubcoreMesh(core_axis_name="c", subcore_axis_name="s")

    @functools.partial(
        pl.kernel,
        out_type=(
            jax.ShapeDtypeStruct((n_tokens, TOPK), jnp.float32),
            jax.ShapeDtypeStruct((n_tokens, TOPK), jnp.int32),
        ),
        mesh=mesh,
        scratch_types=[
            pltpu.VMEM((tok_per_w, n_experts), jnp.float32),
            pltpu.VMEM((tok_per_w + 2, TOPK), jnp.float32),
            pltpu.VMEM((tok_per_w + 2, TOPK), jnp.int32),
            pltpu.VMEM((n_experts,), jnp.float32),
        ],
        compiler_params=pltpu.CompilerParams(needs_layout_passes=False),
    )
    def sc_topk(biased_hbm, bias_hbm, ts_hbm, se_hbm, bs_v, ts_v, se_v, bias_v):
        wid = lax.axis_index("s") * nc + lax.axis_index("c")
        base = wid * tok_per_w
        pltpu.sync_copy(biased_hbm.at[pl.ds(base, tok_per_w)], bs_v)
        pltpu.sync_copy(bias_hbm, bias_v)
        lane_iota = lax.iota(jnp.int32, lanes)
        mask8 = lane_iota < TOPK

        def merge(a, b):
            ak, av = a
            bk, bv = b
            rk = lax.rev(bk, (0,))
            rv = lax.rev(bv, (0,))
            keep_a = ak >= rk
            lk = jnp.where(keep_a, ak, rk)
            lv = jnp.where(keep_a, av, rv)
            return plsc.sort_key_val(lk, lv, descending=True)

        @plsc.parallel_loop(0, tok_per_w, step=1, unroll=4)
        def token_body(t):
            sorted_blks = [
                plsc.sort_key_val(
                    bs_v[t, pl.ds(b * lanes, lanes)],
                    lane_iota + b * lanes,
                    descending=True,
                )
                for b in range(nblk)
            ]
            while len(sorted_blks) > 1:
                sorted_blks = [
                    merge(sorted_blks[i], sorted_blks[i + 1])
                    for i in range(0, len(sorted_blks), 2)
                ]
            mk, mv = sorted_blks[0]
            raw = jnp.where(mask8, mk - plsc.load_gather(bias_v, [mv]), 0.0)
            denom = jnp.broadcast_to(jnp.sum(raw) + 1e-20, raw.shape)
            out = raw / denom
            trow = jnp.full((lanes,), 0, jnp.int32) + t
            plsc.store_scatter(ts_v, [trow, lane_iota], out, mask=mask8)
            plsc.store_scatter(se_v, [trow, lane_iota], mv, mask=mask8)

        pltpu.sync_copy(
            ts_v.at[pl.ds(0, tok_per_w)], ts_hbm.at[pl.ds(base, tok_per_w)]
        )
        pltpu.sync_copy(
            se_v.at[pl.ds(0, tok_per_w)], se_hbm.at[pl.ds(base, tok_per_w)]
        )

    return sc_topk


def kernel(hidden_states, expert_bias, W):
    b, s, h = hidden_states.shape
    n = b * s
    e = W.shape[1]
    x = hidden_states.reshape(n, h)
    biased = _tc_biased_scores(x, W, expert_bias.reshape(1, e))
    sc_topk = _make_sc_topk(n, e)
    return sc_topk(biased, expert_bias)


# speedup vs baseline: 1.2308x; 1.2308x over previous
import functools

import jax
import jax.numpy as jnp
from jax import lax
from jax.experimental import pallas as pl
from jax.experimental.pallas import tpu as pltpu
from jax.experimental.pallas import tpu_sc as plsc

TOPK = 8


def _tc_body(x_ref, wt_ref, b_ref, out_ref):
    logits = lax.dot_general(
        x_ref[...],
        wt_ref[...],
        (((1,), (1,)), ((), ())),
        preferred_element_type=jnp.float32,
    )
    out_ref[...] = jax.nn.sigmoid(logits) + b_ref[...]


@functools.partial(jax.jit, static_argnames=("block_t",))
def _tc_biased_scores(x, wt, bias_2d, block_t=1024):
    n, h = x.shape
    e = wt.shape[0]
    return pl.pallas_call(
        _tc_body,
        grid=(n // block_t,),
        in_specs=[
            pl.BlockSpec((block_t, h), lambda i: (i, 0)),
            pl.BlockSpec((e, h), lambda i: (0, 0)),
            pl.BlockSpec((1, e), lambda i: (0, 0)),
        ],
        out_specs=pl.BlockSpec((block_t, e), lambda i: (i, 0)),
        out_shape=jax.ShapeDtypeStruct((n, e), jnp.float32),
    )(x, wt, bias_2d)


def _make_sc_topk(n_tokens, n_experts):
    info = plsc.get_sparse_core_info()
    nc, ns, lanes = info.num_cores, info.num_subcores, info.num_lanes
    nw = nc * ns
    tok_per_w = n_tokens // nw
    nblk = n_experts // lanes
    mesh = plsc.VectorSubcoreMesh(core_axis_name="c", subcore_axis_name="s")

    @functools.partial(
        pl.kernel,
        out_type=(
            jax.ShapeDtypeStruct((TOPK, n_tokens), jnp.float32),
            jax.ShapeDtypeStruct((TOPK, n_tokens), jnp.int32),
        ),
        mesh=mesh,
        scratch_types=[
            pltpu.VMEM((tok_per_w, n_experts), jnp.float32),
            pltpu.VMEM((TOPK, tok_per_w + 1), jnp.float32),
            pltpu.VMEM((TOPK, tok_per_w + 1), jnp.int32),
            pltpu.VMEM((n_experts,), jnp.float32),
        ],
        compiler_params=pltpu.CompilerParams(needs_layout_passes=False),
    )
    def sc_topk(biased_hbm, bias_hbm, ts_hbm, se_hbm, bs_v, ts_v, se_v, bias_v):
        wid = lax.axis_index("s") * nc + lax.axis_index("c")
        base = wid * tok_per_w
        pltpu.sync_copy(biased_hbm.at[pl.ds(base, tok_per_w)], bs_v)
        pltpu.sync_copy(bias_hbm, bias_v)
        lane_iota = lax.iota(jnp.int32, lanes)
        mask8 = lane_iota < TOPK

        def merge(a, b):
            ak, av = a
            bk, bv = b
            rk = lax.rev(bk, (0,))
            rv = lax.rev(bv, (0,))
            keep_a = ak >= rk
            lk = jnp.where(keep_a, ak, rk)
            lv = jnp.where(keep_a, av, rv)
            return plsc.sort_key_val(lk, lv, descending=True)

        @plsc.parallel_loop(0, tok_per_w, step=1, unroll=4)
        def token_body(t):
            sorted_blks = [
                plsc.sort_key_val(
                    bs_v[t, pl.ds(b * lanes, lanes)],
                    lane_iota + b * lanes,
                    descending=True,
                )
                for b in range(nblk)
            ]
            while len(sorted_blks) > 1:
                sorted_blks = [
                    merge(sorted_blks[i], sorted_blks[i + 1])
                    for i in range(0, len(sorted_blks), 2)
                ]
            mk, mv = sorted_blks[0]
            raw = jnp.where(mask8, mk - plsc.load_gather(bias_v, [mv]), 0.0)
            denom = jnp.broadcast_to(jnp.sum(raw) + 1e-20, raw.shape)
            out = raw / denom
            tcol = jnp.full((lanes,), 0, jnp.int32) + t
            plsc.store_scatter(ts_v, [lane_iota, tcol], out, mask=mask8)
            plsc.store_scatter(se_v, [lane_iota, tcol], mv, mask=mask8)

        pltpu.sync_copy(
            ts_v.at[:, pl.ds(0, tok_per_w)], ts_hbm.at[:, pl.ds(base, tok_per_w)]
        )
        pltpu.sync_copy(
            se_v.at[:, pl.ds(0, tok_per_w)], se_hbm.at[:, pl.ds(base, tok_per_w)]
        )

    return sc_topk


def kernel(hidden_states, expert_bias, W):
    b, s, h = hidden_states.shape
    n = b * s
    e = W.shape[1]
    x = hidden_states.reshape(n, h)
    biased = _tc_biased_scores(x, W.T, expert_bias.reshape(1, e))
    sc_topk = _make_sc_topk(n, e)
    ts_t, se_t = sc_topk(biased, expert_bias)
    return ts_t.T, se_t.T


# speedup vs baseline: 1.2335x; 1.0021x over previous
import functools

import jax
import jax.numpy as jnp
from jax import lax
from jax.experimental import pallas as pl
from jax.experimental.pallas import tpu as pltpu
from jax.experimental.pallas import tpu_sc as plsc

TOPK = 8


def _tc_body(x_ref, wt_ref, b_ref, out_ref):
    logits = lax.dot_general(
        x_ref[...],
        wt_ref[...],
        (((1,), (1,)), ((), ())),
        preferred_element_type=jnp.float32,
    )
    out_ref[...] = jax.nn.sigmoid(logits) + b_ref[...]


@functools.partial(jax.jit, static_argnames=("block_t",))
def _tc_biased_scores(x, wt, bias_2d, block_t=1024):
    n, h = x.shape
    e = wt.shape[0]
    return pl.pallas_call(
        _tc_body,
        grid=(n // block_t,),
        in_specs=[
            pl.BlockSpec((block_t, h), lambda i: (i, 0)),
            pl.BlockSpec((e, h), lambda i: (0, 0)),
            pl.BlockSpec((1, e), lambda i: (0, 0)),
        ],
        out_specs=pl.BlockSpec((block_t, e), lambda i: (i, 0)),
        out_shape=jax.ShapeDtypeStruct((n, e), jnp.float32),
    )(x, wt, bias_2d)


def _make_sc_topk(n_tokens, n_experts):
    info = plsc.get_sparse_core_info()
    nc, ns, lanes = info.num_cores, info.num_subcores, info.num_lanes
    nw = nc * ns
    tok_per_w = n_tokens // nw
    nblk = n_experts // lanes
    mesh = plsc.VectorSubcoreMesh(core_axis_name="c", subcore_axis_name="s")

    @functools.partial(
        pl.kernel,
        out_type=(
            jax.ShapeDtypeStruct((TOPK, n_tokens), jnp.float32),
            jax.ShapeDtypeStruct((TOPK, n_tokens), jnp.int32),
        ),
        mesh=mesh,
        scratch_types=[
            pltpu.VMEM((tok_per_w, n_experts), jnp.float32),
            pltpu.VMEM((TOPK, tok_per_w + 1), jnp.float32),
            pltpu.VMEM((TOPK, tok_per_w + 1), jnp.int32),
            pltpu.VMEM((n_experts,), jnp.float32),
        ],
        compiler_params=pltpu.CompilerParams(needs_layout_passes=False),
    )
    def sc_topk(biased_hbm, bias_hbm, ts_hbm, se_hbm, bs_v, ts_v, se_v, bias_v):
        wid = lax.axis_index("s") * nc + lax.axis_index("c")
        base = wid * tok_per_w
        pltpu.sync_copy(biased_hbm.at[pl.ds(base, tok_per_w)], bs_v)
        pltpu.sync_copy(bias_hbm, bias_v)
        lane_iota = lax.iota(jnp.int32, lanes)
        mask8 = lane_iota < TOPK

        def merge(a, b):
            ak, av = a
            bk, bv = b
            rk = lax.rev(bk, (0,))
            rv = lax.rev(bv, (0,))
            keep_a = ak >= rk
            lk = jnp.where(keep_a, ak, rk)
            lv = jnp.where(keep_a, av, rv)
            return plsc.sort_key_val(lk, lv, descending=True)

        @plsc.parallel_loop(0, tok_per_w, step=1, unroll=2)
        def token_body(t):
            sorted_blks = [
                plsc.sort_key_val(
                    bs_v[t, pl.ds(b * lanes, lanes)],
                    lane_iota + b * lanes,
                    descending=True,
                )
                for b in range(nblk)
            ]
            while len(sorted_blks) > 1:
                sorted_blks = [
                    merge(sorted_blks[i], sorted_blks[i + 1])
                    for i in range(0, len(sorted_blks), 2)
                ]
            mk, mv = sorted_blks[0]
            raw = jnp.where(mask8, mk - plsc.load_gather(bias_v, [mv]), 0.0)
            denom = jnp.broadcast_to(jnp.sum(raw) + 1e-20, raw.shape)
            out = raw / denom
            tcol = jnp.full((lanes,), 0, jnp.int32) + t
            plsc.store_scatter(ts_v, [lane_iota, tcol], out, mask=mask8)
            plsc.store_scatter(se_v, [lane_iota, tcol], mv, mask=mask8)

        pltpu.sync_copy(
            ts_v.at[:, pl.ds(0, tok_per_w)], ts_hbm.at[:, pl.ds(base, tok_per_w)]
        )
        pltpu.sync_copy(
            se_v.at[:, pl.ds(0, tok_per_w)], se_hbm.at[:, pl.ds(base, tok_per_w)]
        )

    return sc_topk


def kernel(hidden_states, expert_bias, W):
    b, s, h = hidden_states.shape
    n = b * s
    e = W.shape[1]
    x = hidden_states.reshape(n, h)
    biased = _tc_biased_scores(x, W.T, expert_bias.reshape(1, e))
    sc_topk = _make_sc_topk(n, e)
    ts_t, se_t = sc_topk(biased, expert_bias)
    return ts_t.T, se_t.T
